# initial kernel scaffold (unmeasured)
import jax
import jax.numpy as jnp
from jax import lax
from jax.experimental import pallas as pl
from jax.experimental.pallas import tpu as pltpu

N_DEV = 4
CAP = 204


def kernel(x, router_W, route_idx, expert_W):
    del router_W
    n_loc, d = x.shape
    e_loc, _, h_out = expert_W.shape
    E = e_loc * N_DEV

    e = route_idx[:, 0]
    onehot = e[:, None] == jnp.arange(E, dtype=e.dtype)[None, :]
    cum = jnp.cumsum(onehot.astype(jnp.int32), axis=0)
    hist = cum[-1, :]
    prior = jnp.where(onehot, (cum - 1).astype(jnp.float32), jnp.float32(1e9))
    hist_pad = (
        jnp.zeros((8, 128), jnp.float32).at[0, :E].set(hist.astype(jnp.float32))
    )
    x_bf = x.astype(jnp.bfloat16)
    w_bf = expert_W.astype(jnp.bfloat16)

    def body(
        x_ref, w_ref, prior_ref, hist_ref, out_ref,
        comm_w, comm_h, wsend, wrecv, hsend, hrecv,
    ):
        p = lax.axis_index("i")
        left = lax.rem(p + N_DEV - 1, N_DEV)
        right = lax.rem(p + 1, N_DEV)

        barrier = pltpu.get_barrier_semaphore()
        for nbr in (left, right):
            pl.semaphore_signal(
                barrier, inc=1, device_id=(nbr,),
                device_id_type=pl.DeviceIdType.MESH,
            )
        pl.semaphore_wait(barrier, 2)

        for hop in range(1, N_DEV):
            src = hist_ref if hop == 1 else comm_h.at[hop - 2]
            rdma = pltpu.make_async_remote_copy(
                src_ref=src,
                dst_ref=comm_h.at[hop - 1],
                send_sem=hsend.at[hop - 1],
                recv_sem=hrecv.at[hop - 1],
                device_id=(right,),
                device_id_type=pl.DeviceIdType.MESH,
            )
            rdma.start()
            rdma.wait()

        off = jnp.zeros((1, E), jnp.float32)
        for s in range(N_DEV - 1):
            o = lax.rem(p + 2 * N_DEV - 1 - s, N_DEV)
            row = comm_h[s, 0:1, 0:E]
            off = off + jnp.where(o < p, row, jnp.zeros_like(row))

        gate = (prior_ref[...] + off) < float(CAP)
        col_ids = lax.broadcasted_iota(jnp.int32, (n_loc, E), 1)
        xv = x_ref[...]

        def block_contrib(origin, load_w):
            base = origin * e_loc
            acc = None
            for k in range(e_loc):
                sel = jnp.sum(
                    jnp.where((col_ids == base + k) & gate, 1.0, 0.0),
                    axis=1, keepdims=True,
                )
                xm = xv * sel.astype(jnp.bfloat16)
                c = jnp.dot(xm, load_w(k), preferred_element_type=jnp.float32)
                acc = c if acc is None else acc + c
            return acc

        out_ref[...] = block_contrib(p, lambda k: w_ref[k])

        for hop in range(1, N_DEV):
            src = w_ref if hop == 1 else comm_w.at[hop - 2]
            rdma = pltpu.make_async_remote_copy(
                src_ref=src,
                dst_ref=comm_w.at[hop - 1],
                send_sem=wsend.at[hop - 1],
                recv_sem=wrecv.at[hop - 1],
                device_id=(right,),
                device_id_type=pl.DeviceIdType.MESH,
            )
            rdma.start()
            rdma.wait()
            origin = lax.rem(p + 2 * N_DEV - hop, N_DEV)
            out_ref[...] += block_contrib(
                origin, lambda k, hop=hop: comm_w[hop - 1, k]
            )

    return pl.pallas_call(
        body,
        out_shape=jax.ShapeDtypeStruct((n_loc, h_out), jnp.float32),
        in_specs=[pl.BlockSpec(memory_space=pltpu.VMEM)] * 4,
        out_specs=pl.BlockSpec(memory_space=pltpu.VMEM),
        scratch_shapes=[
            pltpu.VMEM((N_DEV - 1, e_loc, d, h_out), jnp.bfloat16),
            pltpu.VMEM((N_DEV - 1, 8, 128), jnp.float32),
            pltpu.SemaphoreType.DMA((N_DEV - 1,)),
            pltpu.SemaphoreType.DMA((N_DEV - 1,)),
            pltpu.SemaphoreType.DMA((N_DEV - 1,)),
            pltpu.SemaphoreType.DMA((N_DEV - 1,)),
        ],
        compiler_params=pltpu.CompilerParams(collective_id=0),
    )(x_bf, w_bf, prior, hist_pad)


# baseline (device time: 396524 ns/iter reference)
import jax
import jax.numpy as jnp
from jax import lax
from jax.experimental import pallas as pl
from jax.experimental.pallas import tpu as pltpu

N_DEV = 4
CAP = 204


def kernel(x, router_W, route_idx, expert_W):
    del router_W
    n_loc, d = x.shape
    e_loc, _, h_out = expert_W.shape
    E = e_loc * N_DEV

    e = route_idx[:, 0]
    onehot = e[:, None] == jnp.arange(E, dtype=e.dtype)[None, :]
    cum = jnp.cumsum(onehot.astype(jnp.int32), axis=0)
    hist = cum[-1, :]
    prior = jnp.where(onehot, (cum - 1).astype(jnp.float32), jnp.float32(1e9))
    hist_pad = (
        jnp.zeros((8, 128), jnp.float32).at[0, :E].set(hist.astype(jnp.float32))
    )
    x_bf = x.astype(jnp.bfloat16)
    w_bf = expert_W.astype(jnp.bfloat16)

    def body(
        x_ref, w_ref, prior_ref, hist_ref, out_ref,
        comm_w, comm_h, wsend, wrecv, hsend, hrecv,
    ):
        p = lax.axis_index("i")
        left = lax.rem(p + N_DEV - 1, N_DEV)
        right = lax.rem(p + 1, N_DEV)

        barrier = pltpu.get_barrier_semaphore()
        for nbr in (left, right):
            pl.semaphore_signal(
                barrier, inc=1, device_id=(nbr,),
                device_id_type=pl.DeviceIdType.MESH,
            )
        pl.semaphore_wait(barrier, 2)

        for hop in range(1, N_DEV):
            src = hist_ref if hop == 1 else comm_h.at[hop - 2]
            rdma = pltpu.make_async_remote_copy(
                src_ref=src,
                dst_ref=comm_h.at[hop - 1],
                send_sem=hsend.at[hop - 1],
                recv_sem=hrecv.at[hop - 1],
                device_id=(right,),
                device_id_type=pl.DeviceIdType.MESH,
            )
            rdma.start()
            rdma.wait()

        off = jnp.zeros((1, E), jnp.float32)
        for s in range(N_DEV - 1):
            o = lax.rem(p + 2 * N_DEV - 1 - s, N_DEV)
            row = comm_h[s, 0:1, 0:E]
            off = off + jnp.where(o < p, row, jnp.zeros_like(row))

        gate = (prior_ref[...] + off) < float(CAP)
        col_ids = lax.broadcasted_iota(jnp.int32, (n_loc, E), 1)
        xv = x_ref[...]

        def block_contrib(origin, load_w):
            base = origin * e_loc
            acc = None
            for k in range(e_loc):
                sel = jnp.sum(
                    jnp.where((col_ids == base + k) & gate, 1.0, 0.0),
                    axis=1, keepdims=True,
                )
                xm = xv * sel.astype(jnp.bfloat16)
                c = jnp.dot(xm, load_w(k), preferred_element_type=jnp.float32)
                acc = c if acc is None else acc + c
            return acc

        out_ref[...] = block_contrib(p, lambda k: w_ref[k])

        for hop in range(1, N_DEV):
            src = w_ref if hop == 1 else comm_w.at[hop - 2]
            rdma = pltpu.make_async_remote_copy(
                src_ref=src,
                dst_ref=comm_w.at[hop - 1],
                send_sem=wsend.at[hop - 1],
                recv_sem=wrecv.at[hop - 1],
                device_id=(right,),
                device_id_type=pl.DeviceIdType.MESH,
            )
            rdma.start()
            rdma.wait()
            origin = lax.rem(p + 2 * N_DEV - hop, N_DEV)
            out_ref[...] += block_contrib(
                origin, lambda k, hop=hop: comm_w[hop - 1, k]
            )

    return pl.pallas_call(
        body,
        out_shape=jax.ShapeDtypeStruct((n_loc, h_out), jnp.float32),
        in_specs=[pl.BlockSpec(memory_space=pltpu.VMEM)] * 4,
        out_specs=pl.BlockSpec(memory_space=pltpu.VMEM),
        scratch_shapes=[
            pltpu.VMEM((N_DEV - 1, e_loc, d, h_out), jnp.bfloat16),
            pltpu.VMEM((N_DEV - 1, 8, 128), jnp.float32),
            pltpu.SemaphoreType.DMA((N_DEV - 1,)),
            pltpu.SemaphoreType.DMA((N_DEV - 1,)),
            pltpu.SemaphoreType.DMA((N_DEV - 1,)),
            pltpu.SemaphoreType.DMA((N_DEV - 1,)),
        ],
        compiler_params=pltpu.CompilerParams(
            collective_id=0, vmem_limit_bytes=100 * 1024 * 1024
        ),
    )(x_bf, w_bf, prior, hist_pad)


# device time: 224599 ns/iter; 1.7655x vs baseline; 1.7655x over previous
import jax
import jax.numpy as jnp
from jax import lax
from jax.experimental import pallas as pl
from jax.experimental.pallas import tpu as pltpu

N_DEV = 4
CAP = 204


def kernel(x, router_W, route_idx, expert_W):
    del router_W
    n_loc, d = x.shape
    e_loc, _, h_out = expert_W.shape
    E = e_loc * N_DEV
    half = h_out // 2

    e = route_idx[:, 0]
    onehot = e[:, None] == jnp.arange(E, dtype=e.dtype)[None, :]
    cum = jnp.cumsum(onehot.astype(jnp.int32), axis=0)
    hist = cum[-1, :]
    prior = jnp.where(onehot, (cum - 1).astype(jnp.float32), jnp.float32(1e9))
    hist_pad = (
        jnp.zeros((8, 128), jnp.float32).at[0, :E].set(hist.astype(jnp.float32))
    )
    x_bf = x.astype(jnp.bfloat16)
    w_bf = expert_W.astype(jnp.bfloat16)
    wa = w_bf[:, :, :half]
    wb = w_bf[:, :, half:]

    def body(
        x_ref, wa_ref, wb_ref, prior_ref, hist_ref, out_ref,
        comm_a, comm_b, comm_h, asend, arecv, bsend, brecv, hsend, hrecv,
        acredit, bcredit,
    ):
        p = lax.axis_index("i")
        left = lax.rem(p + N_DEV - 1, N_DEV)
        right = lax.rem(p + 1, N_DEV)

        barrier = pltpu.get_barrier_semaphore()
        for nbr in (left, right):
            pl.semaphore_signal(
                barrier, inc=1, device_id=(nbr,),
                device_id_type=pl.DeviceIdType.MESH,
            )
        pl.semaphore_wait(barrier, 2)

        def ring_copy(src, comm, ssem, rsem, hop, dest, n_slots=2):
            slot = (hop - 1) % n_slots
            return pltpu.make_async_remote_copy(
                src_ref=src,
                dst_ref=comm.at[slot],
                send_sem=ssem.at[slot],
                recv_sem=rsem.at[slot],
                device_id=(dest,),
                device_id_type=pl.DeviceIdType.MESH,
            )

        ra = {1: ring_copy(wa_ref, comm_a, asend, arecv, 1, right)}
        rb = {1: ring_copy(wb_ref, comm_b, bsend, brecv, 1, left)}
        ra[1].start()
        rb[1].start()

        for hop in range(1, N_DEV):
            src = hist_ref if hop == 1 else comm_h.at[hop - 2]
            rdma = ring_copy(src, comm_h, hsend, hrecv, hop, right, n_slots=3)
            rdma.start()
            rdma.wait()

        off = jnp.zeros((1, E), jnp.float32)
        for s in range(N_DEV - 1):
            o = lax.rem(p + 2 * N_DEV - 1 - s, N_DEV)
            row = comm_h[s, 0:1, 0:E]
            off = off + jnp.where(o < p, row, jnp.zeros_like(row))

        gate = (prior_ref[...] + off) < float(CAP)
        col_ids = lax.broadcasted_iota(jnp.int32, (n_loc, E), 1)
        xv = x_ref[...]

        def block_contrib(origin, load_w):
            base = origin * e_loc
            acc = None
            for k in range(e_loc):
                sel = jnp.sum(
                    jnp.where((col_ids == base + k) & gate, 1.0, 0.0),
                    axis=1, keepdims=True,
                )
                xm = xv * sel.astype(jnp.bfloat16)
                c = jnp.dot(xm, load_w(k), preferred_element_type=jnp.float32)
                acc = c if acc is None else acc + c
            return acc

        out_ref[:, 0:half] = block_contrib(p, lambda k: wa_ref[k])
        out_ref[:, half:h_out] = block_contrib(p, lambda k: wb_ref[k])

        for hop in range(1, N_DEV):
            slot = (hop - 1) % 2
            ra[hop].wait()
            rb[hop].wait()
            if hop < N_DEV - 1:
                if hop + 1 == 3:
                    pl.semaphore_wait(acredit, 1)
                ra[hop + 1] = ring_copy(
                    comm_a.at[slot], comm_a, asend, arecv, hop + 1, right
                )
                ra[hop + 1].start()
                if hop + 1 == 3:
                    pl.semaphore_wait(bcredit, 1)
                rb[hop + 1] = ring_copy(
                    comm_b.at[slot], comm_b, bsend, brecv, hop + 1, left
                )
                rb[hop + 1].start()
            o_r = lax.rem(p + 2 * N_DEV - hop, N_DEV)
            o_l = lax.rem(p + hop, N_DEV)
            out_ref[:, 0:half] += block_contrib(
                o_r, lambda k, slot=slot: comm_a[slot, k]
            )
            out_ref[:, half:h_out] += block_contrib(
                o_l, lambda k, slot=slot: comm_b[slot, k]
            )
            if hop == 1:
                pl.semaphore_signal(
                    acredit, inc=1, device_id=(left,),
                    device_id_type=pl.DeviceIdType.MESH,
                )
                pl.semaphore_signal(
                    bcredit, inc=1, device_id=(right,),
                    device_id_type=pl.DeviceIdType.MESH,
                )

    return pl.pallas_call(
        body,
        out_shape=jax.ShapeDtypeStruct((n_loc, h_out), jnp.float32),
        in_specs=[pl.BlockSpec(memory_space=pltpu.VMEM)] * 5,
        out_specs=pl.BlockSpec(memory_space=pltpu.VMEM),
        scratch_shapes=[
            pltpu.VMEM((2, e_loc, d, half), jnp.bfloat16),
            pltpu.VMEM((2, e_loc, d, half), jnp.bfloat16),
            pltpu.VMEM((N_DEV - 1, 8, 128), jnp.float32),
            pltpu.SemaphoreType.DMA((2,)),
            pltpu.SemaphoreType.DMA((2,)),
            pltpu.SemaphoreType.DMA((2,)),
            pltpu.SemaphoreType.DMA((2,)),
            pltpu.SemaphoreType.DMA((N_DEV - 1,)),
            pltpu.SemaphoreType.DMA((N_DEV - 1,)),
            pltpu.SemaphoreType.REGULAR,
            pltpu.SemaphoreType.REGULAR,
        ],
        compiler_params=pltpu.CompilerParams(
            collective_id=0, vmem_limit_bytes=100 * 1024 * 1024
        ),
    )(x_bf, wa, wb, prior, hist_pad)


# device time: 212211 ns/iter; 1.8685x vs baseline; 1.0584x over previous
import jax
import jax.numpy as jnp
from jax import lax
from jax.experimental import pallas as pl
from jax.experimental.pallas import tpu as pltpu

N_DEV = 4
CAP = 204


def kernel(x, router_W, route_idx, expert_W):
    del router_W
    n_loc, d = x.shape
    e_loc, _, h_out = expert_W.shape
    E = e_loc * N_DEV
    half = h_out // 2

    x_bf = x.astype(jnp.bfloat16)
    wa = expert_W[:, :, :half].astype(jnp.bfloat16)
    wb = expert_W[:, :, half:].astype(jnp.bfloat16)

    def body(
        x_ref, wa_ref, wb_ref, ridx_ref, out_ref,
        comm_a, comm_b, comm_h, hstage, asend, arecv, bsend, brecv,
        hsend, hrecv, acredit, bcredit,
    ):
        p = lax.axis_index("i")
        left = lax.rem(p + N_DEV - 1, N_DEV)
        right = lax.rem(p + 1, N_DEV)

        barrier = pltpu.get_barrier_semaphore()
        for nbr in (left, right):
            pl.semaphore_signal(
                barrier, inc=1, device_id=(nbr,),
                device_id_type=pl.DeviceIdType.MESH,
            )
        pl.semaphore_wait(barrier, 2)

        def ring_copy(src, comm, ssem, rsem, hop, dest, n_slots=2):
            slot = (hop - 1) % n_slots
            return pltpu.make_async_remote_copy(
                src_ref=src,
                dst_ref=comm.at[slot],
                send_sem=ssem.at[slot],
                recv_sem=rsem.at[slot],
                device_id=(dest,),
                device_id_type=pl.DeviceIdType.MESH,
            )

        ra = {1: ring_copy(wa_ref, comm_a, asend, arecv, 1, right)}
        rb = {1: ring_copy(wb_ref, comm_b, bsend, brecv, 1, left)}
        ra[1].start()
        rb[1].start()

        col_ids = lax.broadcasted_iota(jnp.int32, (n_loc, E), 1)
        onehot = ridx_ref[...] == col_ids
        oh_f = jnp.where(onehot, 1.0, 0.0).astype(jnp.float32)
        hstage[0:1, 0:E] = jnp.sum(oh_f, axis=0, keepdims=True)

        h1 = ring_copy(hstage, comm_h, hsend, hrecv, 1, right, n_slots=3)
        h1.start()

        cum = oh_f
        sh = 1
        while sh < n_loc:
            cum = cum + jnp.concatenate(
                [jnp.zeros((sh, E), jnp.float32), cum[: n_loc - sh, :]], axis=0
            )
            sh *= 2

        h1.wait()
        for hop in range(2, N_DEV):
            rdma = ring_copy(
                comm_h.at[hop - 2], comm_h, hsend, hrecv, hop, right, n_slots=3
            )
            rdma.start()
            rdma.wait()

        off = jnp.zeros((1, E), jnp.float32)
        for s in range(N_DEV - 1):
            o = lax.rem(p + 2 * N_DEV - 1 - s, N_DEV)
            row = comm_h[s, 0:1, 0:E]
            off = off + jnp.where(o < p, row, jnp.zeros_like(row))

        gate = onehot & ((cum + off) <= float(CAP))
        xv = x_ref[...]

        def block_contrib(origin, load_w):
            base = origin * e_loc
            acc = None
            for k in range(e_loc):
                sel = jnp.sum(
                    jnp.where((col_ids == base + k) & gate, 1.0, 0.0),
                    axis=1, keepdims=True,
                )
                xm = xv * sel.astype(jnp.bfloat16)
                c = jnp.dot(xm, load_w(k), preferred_element_type=jnp.float32)
                acc = c if acc is None else acc + c
            return acc

        out_ref[:, 0:half] = block_contrib(p, lambda k: wa_ref[k])
        out_ref[:, half:h_out] = block_contrib(p, lambda k: wb_ref[k])

        for hop in range(1, N_DEV):
            slot = (hop - 1) % 2
            ra[hop].wait()
            rb[hop].wait()
            if hop < N_DEV - 1:
                if hop + 1 == 3:
                    pl.semaphore_wait(acredit, 1)
                ra[hop + 1] = ring_copy(
                    comm_a.at[slot], comm_a, asend, arecv, hop + 1, right
                )
                ra[hop + 1].start()
                if hop + 1 == 3:
                    pl.semaphore_wait(bcredit, 1)
                rb[hop + 1] = ring_copy(
                    comm_b.at[slot], comm_b, bsend, brecv, hop + 1, left
                )
                rb[hop + 1].start()
            o_r = lax.rem(p + 2 * N_DEV - hop, N_DEV)
            o_l = lax.rem(p + hop, N_DEV)
            out_ref[:, 0:half] += block_contrib(
                o_r, lambda k, slot=slot: comm_a[slot, k]
            )
            out_ref[:, half:h_out] += block_contrib(
                o_l, lambda k, slot=slot: comm_b[slot, k]
            )
            if hop == 1:
                pl.semaphore_signal(
                    acredit, inc=1, device_id=(left,),
                    device_id_type=pl.DeviceIdType.MESH,
                )
                pl.semaphore_signal(
                    bcredit, inc=1, device_id=(right,),
                    device_id_type=pl.DeviceIdType.MESH,
                )

    return pl.pallas_call(
        body,
        out_shape=jax.ShapeDtypeStruct((n_loc, h_out), jnp.float32),
        in_specs=[pl.BlockSpec(memory_space=pltpu.VMEM)] * 4,
        out_specs=pl.BlockSpec(memory_space=pltpu.VMEM),
        scratch_shapes=[
            pltpu.VMEM((2, e_loc, d, half), jnp.bfloat16),
            pltpu.VMEM((2, e_loc, d, half), jnp.bfloat16),
            pltpu.VMEM((N_DEV - 1, 8, 128), jnp.float32),
            pltpu.VMEM((8, 128), jnp.float32),
            pltpu.SemaphoreType.DMA((2,)),
            pltpu.SemaphoreType.DMA((2,)),
            pltpu.SemaphoreType.DMA((2,)),
            pltpu.SemaphoreType.DMA((2,)),
            pltpu.SemaphoreType.DMA((N_DEV - 1,)),
            pltpu.SemaphoreType.DMA((N_DEV - 1,)),
            pltpu.SemaphoreType.REGULAR,
            pltpu.SemaphoreType.REGULAR,
        ],
        compiler_params=pltpu.CompilerParams(
            collective_id=0, vmem_limit_bytes=100 * 1024 * 1024
        ),
    )(x_bf, wa, wb, route_idx)
